# Initial kernel scaffold; baseline (speedup 1.0000x reference)
import jax
import jax.numpy as jnp
from jax.experimental import pallas as pl


def kernel(h_doc, h_word, h_topic, wt_src, wt_dst, ww_src, ww_dst, wd_src, wd_dst, td_src, td_dst, tt_src, tt_dst, W_wt, b_wt, W_ww, b_ww, W_wd, b_wd, W_td, b_td, W_tt, b_tt):
    raise NotImplementedError("write your pallas kernel here")


# speedup vs baseline: 26.5244x; 26.5244x over previous
import functools

import jax
import jax.numpy as jnp
from jax import lax
from jax.experimental import pallas as pl
from jax.experimental.pallas import tpu as pltpu
from jax.experimental.pallas import tpu_sc as plsc

N_WORD = 100000
N_DOC = 10000
N_TOPIC = 512
D = 5
DP = 8
NC, NS = 2, 16
NW = NC * NS
WIN = 1024
PADR = 64

NWP = 100096
NDP = 10240
NTP = 768


def _pad_edges(src, dst, n_src, n_dst):
    e = src.shape[0]
    step = NW * WIN
    e_pad = ((e + step - 1) // step) * step
    pad = e_pad - e
    if pad:
        r = jnp.arange(pad, dtype=jnp.int32) % PADR
        src = jnp.concatenate([src.astype(jnp.int32), n_src + r])
        dst = jnp.concatenate([dst.astype(jnp.int32), n_dst + r])
    else:
        src = src.astype(jnp.int32)
        dst = dst.astype(jnp.int32)
    return src, dst, e_pad


def _pad_feats(h, n_pad):
    n = h.shape[0]
    hp = jnp.zeros((n_pad, DP), jnp.float32)
    hp = hp.at[:n, :D].set(h)
    hp = hp.at[:n, D].set(1.0)
    return hp


_DEG_ACC = (0, 0, 0, 1, 2)
_DEG_SEQ = (0, 1, 2, 0, 1)


def _deg_body(npads, e_pads, *refs):
    n_rel = len(e_pads)
    srcs = refs[:n_rel]
    zeros_hbm, ones_hbm = refs[n_rel], refs[n_rel + 1]
    outs = refs[n_rel + 2:2 * n_rel + 2]
    idx_v, ones_v, sem = refs[2 * n_rel + 2:2 * n_rel + 5]
    accs = refs[2 * n_rel + 5:]

    c = lax.axis_index("c")
    s = lax.axis_index("s")
    wid = s * NC + c

    for acc, npad in zip(accs, (npads[0], npads[3], npads[4])):
        sl = npad // NS
        pltpu.sync_copy(zeros_hbm.at[pl.ds(0, sl)], acc.at[pl.ds(s * sl, sl)])
    pltpu.sync_copy(ones_hbm, ones_v)
    plsc.subcore_barrier()

    n_seq = max(_DEG_SEQ) + 1
    for step in range(n_seq):
        rels = [r for r in range(n_rel) if _DEG_SEQ[r] == step]
        for r in rels:
            src, acc, e_pad = srcs[r], accs[_DEG_ACC[r]], e_pads[r]
            k = e_pad // (NW * WIN)
            base = wid * k * WIN

            def body(i, _, src=src, acc=acc, base=base):
                b = base + i * WIN
                pltpu.sync_copy(src.at[pl.ds(b, WIN)], idx_v)
                pltpu.sync_copy(ones_v, acc.at[idx_v], add=True)
                return 0

            lax.fori_loop(0, k, body, 0)
        plsc.subcore_barrier()
        for r in rels:
            acc, out, npad = accs[_DEG_ACC[r]], outs[r], npads[r]
            sl = npad // NS
            pltpu.sync_copy(acc.at[pl.ds(s * sl, sl)],
                            out.at[c, pl.ds(s * sl, sl)])
            if _DEG_ACC[r] == 0 and step < n_seq - 1:
                pltpu.sync_copy(zeros_hbm.at[pl.ds(0, sl)],
                                acc.at[pl.ds(s * sl, sl)])
        plsc.subcore_barrier()


def _degrees(srcs, npads, e_pads):
    zeros_hbm = jnp.zeros((NWP // NS, DP), jnp.float32)
    ones_hbm = jnp.ones((WIN, DP), jnp.float32)
    mesh = plsc.VectorSubcoreMesh(core_axis_name="c", subcore_axis_name="s")
    body = functools.partial(_deg_body, npads, e_pads)
    return pl.kernel(
        body,
        out_type=tuple(
            jax.ShapeDtypeStruct((NC, npad, DP), jnp.float32)
            for npad in npads
        ),
        mesh=mesh,
        compiler_params=pltpu.CompilerParams(use_tc_tiling_on_sc=False),
        scratch_types=[
            pltpu.VMEM((WIN,), jnp.int32),
            pltpu.VMEM((WIN, DP), jnp.float32),
            pltpu.SemaphoreType.DMA,
        ] + [pltpu.VMEM_SHARED((npad, DP), jnp.float32)
             for npad in (npads[0], npads[3], npads[4])],
    )(*srcs, zeros_hbm, ones_hbm)


def _tables_body(hp, *refs):
    n_rel = len(refs) // 2
    degs = refs[:n_rel]
    outs = refs[n_rel:]
    h = hp[...]
    cols = lax.broadcasted_iota(jnp.int32, h.shape, 1)
    for d_ref, o_ref in zip(degs, outs):
        deg = (d_ref[0] + d_ref[1])[:, 0:1]
        scale = lax.rsqrt(jnp.maximum(deg, 1.0))
        o_ref[...] = jnp.where(cols < D, h * scale, h)


def _tables(hp, degs, npad, bn):
    n_rel = len(degs)
    grid = npad // bn
    return pl.pallas_call(
        _tables_body,
        grid=(grid,),
        in_specs=[pl.BlockSpec((bn, DP), lambda i: (i, 0))] + [
            pl.BlockSpec((NC, bn, DP), lambda i: (0, i, 0))
            for _ in range(n_rel)
        ],
        out_specs=[pl.BlockSpec((bn, DP), lambda i: (i, 0))
                   for _ in range(n_rel)],
        out_shape=[jax.ShapeDtypeStruct((npad, DP), jnp.float32)
                   for _ in range(n_rel)],
    )(hp, *degs)


def _main_body(npads, e_pads, acc_map, *refs):
    n_rel = len(e_pads)
    tabs = refs[:n_rel]
    srcs = refs[n_rel:2 * n_rel]
    dsts = refs[2 * n_rel:3 * n_rel]
    zeros_hbm = refs[3 * n_rel]
    outs = refs[3 * n_rel + 1:4 * n_rel + 1]
    idx_s, idx_d, rows_v, sem = refs[4 * n_rel + 1:4 * n_rel + 5]
    accs = refs[4 * n_rel + 5:]

    c = lax.axis_index("c")
    s = lax.axis_index("s")
    wid = s * NC + c

    for acc, npad in zip(accs, npads):
        sl = npad // NS
        pltpu.sync_copy(zeros_hbm.at[pl.ds(0, sl)], acc.at[pl.ds(s * sl, sl)])
    plsc.subcore_barrier()

    for r in range(n_rel):
        tab, src, dst, e_pad = tabs[r], srcs[r], dsts[r], e_pads[r]
        acc = accs[acc_map[r]]
        k = e_pad // (NW * WIN)
        base = wid * k * WIN

        def body(i, _, tab=tab, src=src, dst=dst, acc=acc, base=base):
            b = base + i * WIN
            pltpu.sync_copy(src.at[pl.ds(b, WIN)], idx_s)
            pltpu.sync_copy(dst.at[pl.ds(b, WIN)], idx_d)
            pltpu.async_copy(tab.at[idx_s], rows_v, sem).wait()
            pltpu.sync_copy(rows_v, acc.at[idx_d], add=True)
            return 0

        lax.fori_loop(0, k, body, 0)

    plsc.subcore_barrier()
    for j, out in enumerate(outs):
        acc, npad = accs[acc_map[j]], npads[acc_map[j]]
        sl = npad // NS
        pltpu.sync_copy(acc.at[pl.ds(s * sl, sl)], out.at[c, pl.ds(s * sl, sl)])


def _main_pass(tabs, srcs, dsts, npads, e_pads, acc_map):
    zeros_hbm = jnp.zeros((NWP // NS, DP), jnp.float32)
    mesh = plsc.VectorSubcoreMesh(core_axis_name="c", subcore_axis_name="s")
    body = functools.partial(_main_body, npads, e_pads, acc_map)
    return pl.kernel(
        body,
        out_type=tuple(
            jax.ShapeDtypeStruct((NC, npads[acc_map[r]], DP), jnp.float32)
            for r in range(len(tabs))
        ),
        mesh=mesh,
        compiler_params=pltpu.CompilerParams(use_tc_tiling_on_sc=False),
        scratch_types=[
            pltpu.VMEM((WIN,), jnp.int32),
            pltpu.VMEM((WIN,), jnp.int32),
            pltpu.VMEM((WIN, DP), jnp.float32),
            pltpu.SemaphoreType.DMA,
        ] + [pltpu.VMEM_SHARED((npad, DP), jnp.float32) for npad in npads],
    )(*tabs, *srcs, *dsts, zeros_hbm)


def _post_body(*refs):
    n_rel = (len(refs) - 1) // 3
    accs = refs[:n_rel]
    ws = refs[n_rel:2 * n_rel]
    bs = refs[2 * n_rel:3 * n_rel]
    out = refs[3 * n_rel]

    def conv(a_ref, w_ref, b_ref):
        sm = a_ref[0] + a_ref[1]
        cnt = sm[:, D:D + 1]
        scale = lax.rsqrt(jnp.maximum(cnt, 1.0))
        agg = sm[:, :D] * scale
        return lax.dot_general(
            agg, w_ref[...], (((1,), (0,)), ((), ())),
            preferred_element_type=jnp.float32,
        ) + b_ref[...][None, :]

    acc = conv(accs[0], ws[0], bs[0])
    for r in range(1, n_rel):
        acc = acc + conv(accs[r], ws[r], bs[r])
    out[...] = acc


def _post(accs, ws, bs, npad, bn):
    n_rel = len(accs)
    grid = npad // bn
    return pl.pallas_call(
        _post_body,
        grid=(grid,),
        in_specs=[pl.BlockSpec((NC, bn, DP), lambda i: (0, i, 0))
                  for _ in range(n_rel)]
        + [pl.BlockSpec((D, D), lambda i: (0, 0)) for _ in range(n_rel)]
        + [pl.BlockSpec((D,), lambda i: (0,)) for _ in range(n_rel)],
        out_specs=pl.BlockSpec((bn, D), lambda i: (i, 0)),
        out_shape=jax.ShapeDtypeStruct((npad, D), jnp.float32),
    )(*accs, *ws, *bs)


def kernel(h_doc, h_word, h_topic, wt_src, wt_dst, ww_src, ww_dst,
           wd_src, wd_dst, td_src, td_dst, tt_src, tt_dst,
           W_wt, b_wt, W_ww, b_ww, W_wd, b_wd, W_td, b_td, W_tt, b_tt):
    ww_s, ww_d, e_ww = _pad_edges(ww_src, ww_dst, N_WORD, N_WORD)
    wd_s, wd_d, e_wd = _pad_edges(wd_src, wd_dst, N_WORD, N_DOC)
    wt_s, wt_d, e_wt = _pad_edges(wt_src, wt_dst, N_WORD, N_TOPIC)
    td_s, td_d, e_td = _pad_edges(td_src, td_dst, N_TOPIC, N_DOC)
    tt_s, tt_d, e_tt = _pad_edges(tt_src, tt_dst, N_TOPIC, N_TOPIC)

    degs = _degrees(
        (ww_s, wd_s, wt_s, td_s, tt_s),
        (NWP, NWP, NWP, NTP, NTP),
        (e_ww, e_wd, e_wt, e_td, e_tt),
    )

    hp_word = _pad_feats(h_word, NWP)
    hp_topic = _pad_feats(h_topic, NTP)
    t_ww, t_wd, t_wt = _tables(hp_word, degs[:3], NWP, NWP // 32)
    t_td, t_tt = _tables(hp_topic, degs[3:], NTP, NTP)

    a_ww, a_wd, a_td, a_wt, a_tt = _main_pass(
        (t_ww, t_wd, t_td, t_wt, t_tt),
        (ww_s, wd_s, td_s, wt_s, tt_s),
        (ww_d, wd_d, td_d, wt_d, tt_d),
        (NWP, NDP, NDP, NTP, NTP),
        (e_ww, e_wd, e_td, e_wt, e_tt),
        (0, 1, 2, 3, 4),
    )

    r_word = _post((a_ww,), (W_ww,), (b_ww,), NWP, NWP // 32)[:N_WORD]
    r_doc = _post((a_wd, a_td), (W_wd, W_td), (b_wd, b_td),
                  NDP, NDP // 4)[:N_DOC]
    r_topic = _post((a_wt, a_tt), (W_wt, W_tt), (b_wt, b_tt),
                    NTP, NTP)[:N_TOPIC]
    return (r_doc, r_word, r_topic)


# speedup vs baseline: 31.3074x; 1.1803x over previous
import functools

import jax
import jax.numpy as jnp
from jax import lax
from jax.experimental import pallas as pl
from jax.experimental.pallas import tpu as pltpu
from jax.experimental.pallas import tpu_sc as plsc

N_WORD = 100000
N_DOC = 10000
N_TOPIC = 512
D = 5
DP = 8
NC, NS = 2, 16
NW = NC * NS
WIN = 1024
PADR = 64

NWP = 100096
NDP = 10240
NTP = 768


def _pad_edges(src, dst, n_src, n_dst):
    e = src.shape[0]
    step = 2 * NW * WIN
    e_pad = ((e + step - 1) // step) * step
    pad = e_pad - e
    if pad:
        r = jnp.arange(pad, dtype=jnp.int32) % PADR
        src = jnp.concatenate([src.astype(jnp.int32), n_src + r])
        dst = jnp.concatenate([dst.astype(jnp.int32), n_dst + r])
    else:
        src = src.astype(jnp.int32)
        dst = dst.astype(jnp.int32)
    return src, dst, e_pad


def _pad_feats(h, n_pad):
    n = h.shape[0]
    hp = jnp.zeros((n_pad, DP), jnp.float32)
    hp = hp.at[:n, :D].set(h)
    hp = hp.at[:n, D].set(1.0)
    return hp


_DEG_ACC = (0, 0, 0, 1, 2)
_DEG_SEQ = (0, 1, 2, 0, 1)


def _deg_body(npads, e_pads, *refs):
    n_rel = len(e_pads)
    srcs = refs[:n_rel]
    zeros_hbm, ones_hbm = refs[n_rel], refs[n_rel + 1]
    outs = refs[n_rel + 2:2 * n_rel + 2]
    idx_a, idx_b, ones_v, sem_ia, sem_ib = refs[2 * n_rel + 2:2 * n_rel + 7]
    accs = refs[2 * n_rel + 7:]

    c = lax.axis_index("c")
    s = lax.axis_index("s")
    wid = s * NC + c

    for acc, npad in zip(accs, (npads[0], npads[3], npads[4])):
        sl = npad // NS
        pltpu.sync_copy(zeros_hbm.at[pl.ds(0, sl)], acc.at[pl.ds(s * sl, sl)])
    pltpu.sync_copy(ones_hbm, ones_v)
    plsc.subcore_barrier()

    n_seq = max(_DEG_SEQ) + 1
    for step in range(n_seq):
        rels = [r for r in range(n_rel) if _DEG_SEQ[r] == step]
        for r in rels:
            src, acc, e_pad = srcs[r], accs[_DEG_ACC[r]], e_pads[r]
            k = e_pad // (NW * WIN)
            kh = k // 2
            base = wid * k * WIN
            pltpu.async_copy(src.at[pl.ds(base, WIN)], idx_a, sem_ia)

            def body(j, _, src=src, acc=acc, base=base, kh=kh):
                oa = base + (2 * j) * WIN
                ob = oa + WIN
                pltpu.make_async_copy(src.at[pl.ds(oa, WIN)], idx_a,
                                      sem_ia).wait()
                pltpu.async_copy(src.at[pl.ds(ob, WIN)], idx_b, sem_ib)
                pltpu.sync_copy(ones_v, acc.at[idx_a], add=True)
                pltpu.make_async_copy(src.at[pl.ds(ob, WIN)], idx_b,
                                      sem_ib).wait()

                @pl.when(j < kh - 1)
                def _():
                    nxt = base + (2 * j + 2) * WIN
                    pltpu.async_copy(src.at[pl.ds(nxt, WIN)], idx_a, sem_ia)

                pltpu.sync_copy(ones_v, acc.at[idx_b], add=True)
                return 0

            lax.fori_loop(0, kh, body, 0)
        plsc.subcore_barrier()
        for r in rels:
            acc, out, npad = accs[_DEG_ACC[r]], outs[r], npads[r]
            sl = npad // NS
            pltpu.sync_copy(acc.at[pl.ds(s * sl, sl)],
                            out.at[c, pl.ds(s * sl, sl)])
            if _DEG_ACC[r] == 0 and step < n_seq - 1:
                pltpu.sync_copy(zeros_hbm.at[pl.ds(0, sl)],
                                acc.at[pl.ds(s * sl, sl)])
        plsc.subcore_barrier()


def _degrees(srcs, npads, e_pads):
    zeros_hbm = jnp.zeros((NWP // NS, DP), jnp.float32)
    ones_hbm = jnp.ones((WIN, DP), jnp.float32)
    mesh = plsc.VectorSubcoreMesh(core_axis_name="c", subcore_axis_name="s")
    body = functools.partial(_deg_body, npads, e_pads)
    return pl.kernel(
        body,
        out_type=tuple(
            jax.ShapeDtypeStruct((NC, npad, DP), jnp.float32)
            for npad in npads
        ),
        mesh=mesh,
        compiler_params=pltpu.CompilerParams(use_tc_tiling_on_sc=False),
        scratch_types=[
            pltpu.VMEM((WIN,), jnp.int32),
            pltpu.VMEM((WIN,), jnp.int32),
            pltpu.VMEM((WIN, DP), jnp.float32),
            pltpu.SemaphoreType.DMA,
            pltpu.SemaphoreType.DMA,
        ] + [pltpu.VMEM_SHARED((npad, DP), jnp.float32)
             for npad in (npads[0], npads[3], npads[4])],
    )(*srcs, zeros_hbm, ones_hbm)


def _tables_body(hp, *refs):
    n_rel = len(refs) // 2
    degs = refs[:n_rel]
    outs = refs[n_rel:]
    h = hp[...]
    cols = lax.broadcasted_iota(jnp.int32, h.shape, 1)
    for d_ref, o_ref in zip(degs, outs):
        deg = (d_ref[0] + d_ref[1])[:, 0:1]
        scale = lax.rsqrt(jnp.maximum(deg, 1.0))
        o_ref[...] = jnp.where(cols < D, h * scale, h)


def _tables(hp, degs, npad, bn):
    n_rel = len(degs)
    grid = npad // bn
    return pl.pallas_call(
        _tables_body,
        grid=(grid,),
        in_specs=[pl.BlockSpec((bn, DP), lambda i: (i, 0))] + [
            pl.BlockSpec((NC, bn, DP), lambda i: (0, i, 0))
            for _ in range(n_rel)
        ],
        out_specs=[pl.BlockSpec((bn, DP), lambda i: (i, 0))
                   for _ in range(n_rel)],
        out_shape=[jax.ShapeDtypeStruct((npad, DP), jnp.float32)
                   for _ in range(n_rel)],
    )(hp, *degs)


def _main_body(npads, e_pads, acc_map, *refs):
    n_rel = len(e_pads)
    tabs = refs[:n_rel]
    srcs = refs[n_rel:2 * n_rel]
    dsts = refs[2 * n_rel:3 * n_rel]
    zeros_hbm = refs[3 * n_rel]
    outs = refs[3 * n_rel + 1:4 * n_rel + 1]
    (idx_sa, idx_sb, idx_da, idx_db, rows_a, rows_b,
     sem_ia, sem_ib, sem_ga, sem_gb) = refs[4 * n_rel + 1:4 * n_rel + 11]
    accs = refs[4 * n_rel + 11:]

    c = lax.axis_index("c")
    s = lax.axis_index("s")
    wid = s * NC + c

    for acc, npad in zip(accs, npads):
        sl = npad // NS
        pltpu.sync_copy(zeros_hbm.at[pl.ds(0, sl)], acc.at[pl.ds(s * sl, sl)])
    plsc.subcore_barrier()

    for r in range(n_rel):
        tab, src, dst, e_pad = tabs[r], srcs[r], dsts[r], e_pads[r]
        acc = accs[acc_map[r]]
        k = e_pad // (NW * WIN)
        kh = k // 2
        base = wid * k * WIN

        pltpu.async_copy(src.at[pl.ds(base, WIN)], idx_sa, sem_ia)
        pltpu.async_copy(dst.at[pl.ds(base, WIN)], idx_da, sem_ia)

        def body(j, _, tab=tab, src=src, dst=dst, acc=acc,
                 base=base, kh=kh):
            oa = base + (2 * j) * WIN
            ob = oa + WIN
            pltpu.make_async_copy(src.at[pl.ds(oa, WIN)], idx_sa,
                                  sem_ia).wait()
            pltpu.make_async_copy(dst.at[pl.ds(oa, WIN)], idx_da,
                                  sem_ia).wait()
            pltpu.async_copy(tab.at[idx_sa], rows_a, sem_ga)

            @pl.when(j > 0)
            def _():
                pltpu.make_async_copy(tab.at[idx_sb], rows_b, sem_gb).wait()
                pltpu.sync_copy(rows_b, acc.at[idx_db], add=True)

            pltpu.async_copy(src.at[pl.ds(ob, WIN)], idx_sb, sem_ib)
            pltpu.async_copy(dst.at[pl.ds(ob, WIN)], idx_db, sem_ib)

            pltpu.make_async_copy(src.at[pl.ds(ob, WIN)], idx_sb,
                                  sem_ib).wait()
            pltpu.make_async_copy(dst.at[pl.ds(ob, WIN)], idx_db,
                                  sem_ib).wait()
            pltpu.async_copy(tab.at[idx_sb], rows_b, sem_gb)
            pltpu.make_async_copy(tab.at[idx_sa], rows_a, sem_ga).wait()
            pltpu.sync_copy(rows_a, acc.at[idx_da], add=True)

            @pl.when(j < kh - 1)
            def _():
                nxt = base + (2 * j + 2) * WIN
                pltpu.async_copy(src.at[pl.ds(nxt, WIN)], idx_sa, sem_ia)
                pltpu.async_copy(dst.at[pl.ds(nxt, WIN)], idx_da, sem_ia)
            return 0

        lax.fori_loop(0, kh, body, 0)
        pltpu.make_async_copy(tab.at[idx_sb], rows_b, sem_gb).wait()
        pltpu.sync_copy(rows_b, acc.at[idx_db], add=True)

    plsc.subcore_barrier()
    for j, out in enumerate(outs):
        acc, npad = accs[acc_map[j]], npads[acc_map[j]]
        sl = npad // NS
        pltpu.sync_copy(acc.at[pl.ds(s * sl, sl)], out.at[c, pl.ds(s * sl, sl)])


def _main_pass(tabs, srcs, dsts, npads, e_pads, acc_map):
    zeros_hbm = jnp.zeros((NWP // NS, DP), jnp.float32)
    mesh = plsc.VectorSubcoreMesh(core_axis_name="c", subcore_axis_name="s")
    body = functools.partial(_main_body, npads, e_pads, acc_map)
    return pl.kernel(
        body,
        out_type=tuple(
            jax.ShapeDtypeStruct((NC, npads[acc_map[r]], DP), jnp.float32)
            for r in range(len(tabs))
        ),
        mesh=mesh,
        compiler_params=pltpu.CompilerParams(use_tc_tiling_on_sc=False),
        scratch_types=[
            pltpu.VMEM((WIN,), jnp.int32),
            pltpu.VMEM((WIN,), jnp.int32),
            pltpu.VMEM((WIN,), jnp.int32),
            pltpu.VMEM((WIN,), jnp.int32),
            pltpu.VMEM((WIN, DP), jnp.float32),
            pltpu.VMEM((WIN, DP), jnp.float32),
            pltpu.SemaphoreType.DMA,
            pltpu.SemaphoreType.DMA,
            pltpu.SemaphoreType.DMA,
            pltpu.SemaphoreType.DMA,
        ] + [pltpu.VMEM_SHARED((npad, DP), jnp.float32) for npad in npads],
    )(*tabs, *srcs, *dsts, zeros_hbm)


def _post_body(*refs):
    n_rel = (len(refs) - 1) // 3
    accs = refs[:n_rel]
    ws = refs[n_rel:2 * n_rel]
    bs = refs[2 * n_rel:3 * n_rel]
    out = refs[3 * n_rel]

    def conv(a_ref, w_ref, b_ref):
        sm = a_ref[0] + a_ref[1]
        cnt = sm[:, D:D + 1]
        scale = lax.rsqrt(jnp.maximum(cnt, 1.0))
        agg = sm[:, :D] * scale
        return lax.dot_general(
            agg, w_ref[...], (((1,), (0,)), ((), ())),
            preferred_element_type=jnp.float32,
        ) + b_ref[...][None, :]

    acc = conv(accs[0], ws[0], bs[0])
    for r in range(1, n_rel):
        acc = acc + conv(accs[r], ws[r], bs[r])
    out[...] = acc


def _post(accs, ws, bs, npad, bn):
    n_rel = len(accs)
    grid = npad // bn
    return pl.pallas_call(
        _post_body,
        grid=(grid,),
        in_specs=[pl.BlockSpec((NC, bn, DP), lambda i: (0, i, 0))
                  for _ in range(n_rel)]
        + [pl.BlockSpec((D, D), lambda i: (0, 0)) for _ in range(n_rel)]
        + [pl.BlockSpec((D,), lambda i: (0,)) for _ in range(n_rel)],
        out_specs=pl.BlockSpec((bn, D), lambda i: (i, 0)),
        out_shape=jax.ShapeDtypeStruct((npad, D), jnp.float32),
    )(*accs, *ws, *bs)


def kernel(h_doc, h_word, h_topic, wt_src, wt_dst, ww_src, ww_dst,
           wd_src, wd_dst, td_src, td_dst, tt_src, tt_dst,
           W_wt, b_wt, W_ww, b_ww, W_wd, b_wd, W_td, b_td, W_tt, b_tt):
    ww_s, ww_d, e_ww = _pad_edges(ww_src, ww_dst, N_WORD, N_WORD)
    wd_s, wd_d, e_wd = _pad_edges(wd_src, wd_dst, N_WORD, N_DOC)
    wt_s, wt_d, e_wt = _pad_edges(wt_src, wt_dst, N_WORD, N_TOPIC)
    td_s, td_d, e_td = _pad_edges(td_src, td_dst, N_TOPIC, N_DOC)
    tt_s, tt_d, e_tt = _pad_edges(tt_src, tt_dst, N_TOPIC, N_TOPIC)

    degs = _degrees(
        (ww_s, wd_s, wt_s, td_s, tt_s),
        (NWP, NWP, NWP, NTP, NTP),
        (e_ww, e_wd, e_wt, e_td, e_tt),
    )

    hp_word = _pad_feats(h_word, NWP)
    hp_topic = _pad_feats(h_topic, NTP)
    t_ww, t_wd, t_wt = _tables(hp_word, degs[:3], NWP, NWP // 32)
    t_td, t_tt = _tables(hp_topic, degs[3:], NTP, NTP)

    a_ww, a_wd, a_td, a_wt, a_tt = _main_pass(
        (t_ww, t_wd, t_td, t_wt, t_tt),
        (ww_s, wd_s, td_s, wt_s, tt_s),
        (ww_d, wd_d, td_d, wt_d, tt_d),
        (NWP, NDP, NDP, NTP, NTP),
        (e_ww, e_wd, e_td, e_wt, e_tt),
        (0, 1, 2, 3, 4),
    )

    r_word = _post((a_ww,), (W_ww,), (b_ww,), NWP, NWP // 32)[:N_WORD]
    r_doc = _post((a_wd, a_td), (W_wd, W_td), (b_wd, b_td),
                  NDP, NDP // 4)[:N_DOC]
    r_topic = _post((a_wt, a_tt), (W_wt, W_tt), (b_wt, b_tt),
                    NTP, NTP)[:N_TOPIC]
    return (r_doc, r_word, r_topic)


# speedup vs baseline: 61.0324x; 1.9495x over previous
import functools

import jax
import jax.numpy as jnp
import numpy as np
from jax import lax
from jax.experimental import pallas as pl
from jax.experimental.pallas import tpu as pltpu
from jax.experimental.pallas import tpu_sc as plsc

N_WORD = 100000
N_DOC = 10000
N_TOPIC = 512
D = 5
DP = 8
NC, NS = 2, 16
NW = NC * NS
WIN = 1024
PADR = 64

NWP = 100096
NDP = 10240
NTP = 768


def _pad_edges(src, dst, n_src, n_dst):
    e = src.shape[0]
    step = 2 * NW * WIN
    e_pad = ((e + step - 1) // step) * step
    pad = e_pad - e
    if pad:
        r = jnp.arange(pad, dtype=jnp.int32) % PADR
        src = jnp.concatenate([src.astype(jnp.int32), n_src + r])
        dst = jnp.concatenate([dst.astype(jnp.int32), n_dst + r])
    else:
        src = src.astype(jnp.int32)
        dst = dst.astype(jnp.int32)
    return src, dst, e_pad


def _pad_feats(h, n_pad):
    n = h.shape[0]
    hp = jnp.zeros((n_pad, DP), jnp.float32)
    hp = hp.at[:n, :D].set(h)
    hp = hp.at[:n, D].set(1.0)
    return hp


_DEG_ACC = (0, 0, 0, 1, 2)
_DEG_SEQ = (0, 1, 2, 0, 1)


def _deg_body(npads, e_pads, *refs):
    n_rel = len(e_pads)
    srcs = refs[:n_rel]
    zeros_hbm, ones_hbm = refs[n_rel], refs[n_rel + 1]
    outs = refs[n_rel + 2:2 * n_rel + 2]
    idx_a, idx_b, ones_v, sem_ia, sem_ib = refs[2 * n_rel + 2:2 * n_rel + 7]
    accs = refs[2 * n_rel + 7:]

    c = lax.axis_index("c")
    s = lax.axis_index("s")
    wid = s * NC + c

    for acc, npad in zip(accs, (npads[0], npads[3], npads[4])):
        sl = npad // NS
        pltpu.sync_copy(zeros_hbm.at[pl.ds(0, sl)], acc.at[pl.ds(s * sl, sl)])
    pltpu.sync_copy(ones_hbm, ones_v)
    plsc.subcore_barrier()

    n_seq = max(_DEG_SEQ) + 1
    for step in range(n_seq):
        rels = [r for r in range(n_rel) if _DEG_SEQ[r] == step]
        for r in rels:
            src, acc, e_pad = srcs[r], accs[_DEG_ACC[r]], e_pads[r]
            k = e_pad // (NW * WIN)
            kh = k // 2
            base = wid * k * WIN
            pltpu.async_copy(src.at[pl.ds(base, WIN)], idx_a, sem_ia)

            def body(j, _, src=src, acc=acc, base=base, kh=kh):
                oa = base + (2 * j) * WIN
                ob = oa + WIN
                pltpu.make_async_copy(src.at[pl.ds(oa, WIN)], idx_a,
                                      sem_ia).wait()
                pltpu.async_copy(src.at[pl.ds(ob, WIN)], idx_b, sem_ib)
                pltpu.sync_copy(ones_v, acc.at[idx_a], add=True)
                pltpu.make_async_copy(src.at[pl.ds(ob, WIN)], idx_b,
                                      sem_ib).wait()

                @pl.when(j < kh - 1)
                def _():
                    nxt = base + (2 * j + 2) * WIN
                    pltpu.async_copy(src.at[pl.ds(nxt, WIN)], idx_a, sem_ia)

                pltpu.sync_copy(ones_v, acc.at[idx_b], add=True)
                return 0

            lax.fori_loop(0, kh, body, 0)
        plsc.subcore_barrier()
        for r in rels:
            acc, out, npad = accs[_DEG_ACC[r]], outs[r], npads[r]
            sl = npad // NS
            pltpu.sync_copy(acc.at[pl.ds(s * sl, sl)],
                            out.at[c, pl.ds(s * sl, sl)])
            if _DEG_ACC[r] == 0 and step < n_seq - 1:
                pltpu.sync_copy(zeros_hbm.at[pl.ds(0, sl)],
                                acc.at[pl.ds(s * sl, sl)])
        plsc.subcore_barrier()


def _degrees(srcs, npads, e_pads):
    zeros_hbm = jnp.zeros((NWP // NS, DP), jnp.float32)
    ones_hbm = jnp.ones((WIN, DP), jnp.float32)
    mesh = plsc.VectorSubcoreMesh(core_axis_name="c", subcore_axis_name="s")
    body = functools.partial(_deg_body, npads, e_pads)
    return pl.kernel(
        body,
        out_type=tuple(
            jax.ShapeDtypeStruct((NC, npad, DP), jnp.float32)
            for npad in npads
        ),
        mesh=mesh,
        compiler_params=pltpu.CompilerParams(use_tc_tiling_on_sc=False),
        scratch_types=[
            pltpu.VMEM((WIN,), jnp.int32),
            pltpu.VMEM((WIN,), jnp.int32),
            pltpu.VMEM((WIN, DP), jnp.float32),
            pltpu.SemaphoreType.DMA,
            pltpu.SemaphoreType.DMA,
        ] + [pltpu.VMEM_SHARED((npad, DP), jnp.float32)
             for npad in (npads[0], npads[3], npads[4])],
    )(*srcs, zeros_hbm, ones_hbm)


def _tables_body(hp, *refs):
    n_rel = len(refs) // 2
    degs = refs[:n_rel]
    outs = refs[n_rel:]
    h = hp[...]
    lane = lax.broadcasted_iota(jnp.int32, h.shape, 1) % DP
    feat = lane < D
    for d_ref, o_ref in zip(degs, outs):
        deg = d_ref[0] + d_ref[1]
        scale = lax.rsqrt(jnp.maximum(deg, 1.0))
        o_ref[...] = jnp.where(feat, h * scale, h)


def _tables(hp, degs, npad16, bn):
    n_rel = len(degs)
    grid = npad16 // bn
    return pl.pallas_call(
        _tables_body,
        grid=(grid,),
        in_specs=[pl.BlockSpec((bn, 128), lambda i: (i, 0))] + [
            pl.BlockSpec((NC, bn, 128), lambda i: (0, i, 0))
            for _ in range(n_rel)
        ],
        out_specs=[pl.BlockSpec((bn, 128), lambda i: (i, 0))
                   for _ in range(n_rel)],
        out_shape=[jax.ShapeDtypeStruct((npad16, 128), jnp.float32)
                   for _ in range(n_rel)],
    )(hp, *degs)


def _main_body(npads, e_pads, acc_map, *refs):
    n_rel = len(e_pads)
    tabs = refs[:n_rel]
    srcs = refs[n_rel:2 * n_rel]
    dsts = refs[2 * n_rel:3 * n_rel]
    zeros_hbm = refs[3 * n_rel]
    outs = refs[3 * n_rel + 1:4 * n_rel + 1]
    (idx_sa, idx_sb, idx_da, idx_db, rows_a, rows_b,
     sem_ia, sem_ib, sem_ga, sem_gb) = refs[4 * n_rel + 1:4 * n_rel + 11]
    accs = refs[4 * n_rel + 11:]

    c = lax.axis_index("c")
    s = lax.axis_index("s")
    wid = s * NC + c

    for acc, npad in zip(accs, npads):
        sl = npad // NS
        pltpu.sync_copy(zeros_hbm.at[pl.ds(0, sl)], acc.at[pl.ds(s * sl, sl)])
    plsc.subcore_barrier()

    for r in range(n_rel):
        tab, src, dst, e_pad = tabs[r], srcs[r], dsts[r], e_pads[r]
        acc = accs[acc_map[r]]
        k = e_pad // (NW * WIN)
        kh = k // 2
        base = wid * k * WIN

        pltpu.async_copy(src.at[pl.ds(base, WIN)], idx_sa, sem_ia)
        pltpu.async_copy(dst.at[pl.ds(base, WIN)], idx_da, sem_ia)

        def body(j, _, tab=tab, src=src, dst=dst, acc=acc,
                 base=base, kh=kh):
            oa = base + (2 * j) * WIN
            ob = oa + WIN
            pltpu.make_async_copy(src.at[pl.ds(oa, WIN)], idx_sa,
                                  sem_ia).wait()
            pltpu.make_async_copy(dst.at[pl.ds(oa, WIN)], idx_da,
                                  sem_ia).wait()
            pltpu.async_copy(tab.at[idx_sa], rows_a, sem_ga)

            @pl.when(j > 0)
            def _():
                pltpu.make_async_copy(tab.at[idx_sb], rows_b, sem_gb).wait()
                pltpu.sync_copy(rows_b, acc.at[idx_db], add=True)

            pltpu.async_copy(src.at[pl.ds(ob, WIN)], idx_sb, sem_ib)
            pltpu.async_copy(dst.at[pl.ds(ob, WIN)], idx_db, sem_ib)

            pltpu.make_async_copy(src.at[pl.ds(ob, WIN)], idx_sb,
                                  sem_ib).wait()
            pltpu.make_async_copy(dst.at[pl.ds(ob, WIN)], idx_db,
                                  sem_ib).wait()
            pltpu.async_copy(tab.at[idx_sb], rows_b, sem_gb)
            pltpu.make_async_copy(tab.at[idx_sa], rows_a, sem_ga).wait()
            pltpu.sync_copy(rows_a, acc.at[idx_da], add=True)

            @pl.when(j < kh - 1)
            def _():
                nxt = base + (2 * j + 2) * WIN
                pltpu.async_copy(src.at[pl.ds(nxt, WIN)], idx_sa, sem_ia)
                pltpu.async_copy(dst.at[pl.ds(nxt, WIN)], idx_da, sem_ia)
            return 0

        lax.fori_loop(0, kh, body, 0)
        pltpu.make_async_copy(tab.at[idx_sb], rows_b, sem_gb).wait()
        pltpu.sync_copy(rows_b, acc.at[idx_db], add=True)

    plsc.subcore_barrier()
    for j, out in enumerate(outs):
        acc, npad = accs[acc_map[j]], npads[acc_map[j]]
        sl = npad // NS
        pltpu.sync_copy(acc.at[pl.ds(s * sl, sl)], out.at[c, pl.ds(s * sl, sl)])


def _main_pass(tabs, srcs, dsts, npads, e_pads, acc_map):
    zeros_hbm = jnp.zeros((NWP // NS, DP), jnp.float32)
    mesh = plsc.VectorSubcoreMesh(core_axis_name="c", subcore_axis_name="s")
    body = functools.partial(_main_body, npads, e_pads, acc_map)
    return pl.kernel(
        body,
        out_type=tuple(
            jax.ShapeDtypeStruct((NC, npads[acc_map[r]], DP), jnp.float32)
            for r in range(len(tabs))
        ),
        mesh=mesh,
        compiler_params=pltpu.CompilerParams(use_tc_tiling_on_sc=False),
        scratch_types=[
            pltpu.VMEM((WIN,), jnp.int32),
            pltpu.VMEM((WIN,), jnp.int32),
            pltpu.VMEM((WIN,), jnp.int32),
            pltpu.VMEM((WIN,), jnp.int32),
            pltpu.VMEM((WIN, DP), jnp.float32),
            pltpu.VMEM((WIN, DP), jnp.float32),
            pltpu.SemaphoreType.DMA,
            pltpu.SemaphoreType.DMA,
            pltpu.SemaphoreType.DMA,
            pltpu.SemaphoreType.DMA,
        ] + [pltpu.VMEM_SHARED((npad, DP), jnp.float32) for npad in npads],
    )(*tabs, *srcs, *dsts, zeros_hbm)


def _post_body(sel, *refs):
    n_rel = (len(refs) - 1) // 3
    accs = refs[:n_rel]
    bws = refs[n_rel:2 * n_rel]
    bias = refs[2 * n_rel:3 * n_rel]
    out = refs[3 * n_rel]

    def conv(a_ref, bw_ref, b_ref):
        sm = a_ref[0] + a_ref[1]
        cnt = lax.dot_general(sm, sel[...], (((1,), (0,)), ((), ())),
                              preferred_element_type=jnp.float32)
        scale = lax.rsqrt(jnp.maximum(cnt, 1.0))
        return lax.dot_general(sm * scale, bw_ref[...],
                               (((1,), (0,)), ((), ())),
                               preferred_element_type=jnp.float32) + b_ref[...]

    acc = conv(accs[0], bws[0], bias[0])
    for r in range(1, n_rel):
        acc = acc + conv(accs[r], bws[r], bias[r])
    out[...] = acc


def _post(accs, bws, bias, sel, npad16, bn):
    n_rel = len(accs)
    grid = npad16 // bn
    return pl.pallas_call(
        functools.partial(_post_body),
        grid=(grid,),
        in_specs=[pl.BlockSpec((128, 128), lambda i: (0, 0))]
        + [pl.BlockSpec((NC, bn, 128), lambda i: (0, i, 0))
           for _ in range(n_rel)]
        + [pl.BlockSpec((128, 128), lambda i: (0, 0)) for _ in range(n_rel)]
        + [pl.BlockSpec((1, 128), lambda i: (0, 0)) for _ in range(n_rel)],
        out_specs=pl.BlockSpec((bn, 128), lambda i: (i, 0)),
        out_shape=jax.ShapeDtypeStruct((npad16, 128), jnp.float32),
    )(sel, *accs, *bws, *bias)


def _pack(x, npad):
    return x.reshape(x.shape[:-2] + (npad // 16, 128))


def _mk_sel():
    s = np.zeros((128, 128), np.float32)
    for r in range(16):
        s[8 * r + D, 8 * r:8 * r + D] = 1.0
    return jnp.asarray(s)


def _mk_bw(w):
    wp = jnp.zeros((DP, DP), jnp.float32).at[:D, :D].set(w)
    return jnp.kron(jnp.eye(16, dtype=jnp.float32), wp)


def _mk_bias(b):
    return jnp.tile(jnp.pad(b, (0, DP - D)), 16)[None, :]


def kernel(h_doc, h_word, h_topic, wt_src, wt_dst, ww_src, ww_dst,
           wd_src, wd_dst, td_src, td_dst, tt_src, tt_dst,
           W_wt, b_wt, W_ww, b_ww, W_wd, b_wd, W_td, b_td, W_tt, b_tt):
    ww_s, ww_d, e_ww = _pad_edges(ww_src, ww_dst, N_WORD, N_WORD)
    wd_s, wd_d, e_wd = _pad_edges(wd_src, wd_dst, N_WORD, N_DOC)
    wt_s, wt_d, e_wt = _pad_edges(wt_src, wt_dst, N_WORD, N_TOPIC)
    td_s, td_d, e_td = _pad_edges(td_src, td_dst, N_TOPIC, N_DOC)
    tt_s, tt_d, e_tt = _pad_edges(tt_src, tt_dst, N_TOPIC, N_TOPIC)

    degs = _degrees(
        (ww_s, wd_s, wt_s, td_s, tt_s),
        (NWP, NWP, NWP, NTP, NTP),
        (e_ww, e_wd, e_wt, e_td, e_tt),
    )
    degs_p = [_pack(d, npad)
              for d, npad in zip(degs, (NWP, NWP, NWP, NTP, NTP))]

    hp_word = _pack(_pad_feats(h_word, NWP), NWP)
    hp_topic = _pack(_pad_feats(h_topic, NTP), NTP)
    tw = _tables(hp_word, degs_p[:3], NWP // 16, NWP // 32)
    tt_tabs = _tables(hp_topic, degs_p[3:], NTP // 16, NTP // 16)
    t_ww, t_wd, t_wt = (t.reshape(NWP, DP) for t in tw)
    t_td, t_tt = (t.reshape(NTP, DP) for t in tt_tabs)

    a_ww, a_wd, a_td, a_wt, a_tt = _main_pass(
        (t_ww, t_wd, t_td, t_wt, t_tt),
        (ww_s, wd_s, td_s, wt_s, tt_s),
        (ww_d, wd_d, td_d, wt_d, tt_d),
        (NWP, NDP, NDP, NTP, NTP),
        (e_ww, e_wd, e_td, e_wt, e_tt),
        (0, 1, 2, 3, 4),
    )

    sel = _mk_sel()
    r_word = _post((_pack(a_ww, NWP),), (_mk_bw(W_ww),), (_mk_bias(b_ww),),
                   sel, NWP // 16, NWP // 32)
    r_doc = _post((_pack(a_wd, NDP), _pack(a_td, NDP)),
                  (_mk_bw(W_wd), _mk_bw(W_td)),
                  (_mk_bias(b_wd), _mk_bias(b_td)),
                  sel, NDP // 16, NDP // 16)
    r_topic = _post((_pack(a_wt, NTP), _pack(a_tt, NTP)),
                    (_mk_bw(W_wt), _mk_bw(W_tt)),
                    (_mk_bias(b_wt), _mk_bias(b_tt)),
                    sel, NTP // 16, NTP // 16)
    r_word = r_word.reshape(NWP, DP)[:N_WORD, :D]
    r_doc = r_doc.reshape(NDP, DP)[:N_DOC, :D]
    r_topic = r_topic.reshape(NTP, DP)[:N_TOPIC, :D]
    return (r_doc, r_word, r_topic)
